# baseline (speedup 1.0000x reference)
import functools

import jax
import jax.numpy as jnp
from jax import lax
from jax.experimental import pallas as pl
from jax.experimental.pallas import tpu as pltpu
from jax.experimental.pallas import tpu_sc as plsc

H = 64
HIST = 200
HPAD = 208
CH = HPAD // 2
LANES = 16
NVD = H // LANES


def _build(B, NC, NS):
  NW = NC * NS
  IPW = B // NW

  mesh = plsc.VectorSubcoreMesh(core_axis_name="c", subcore_axis_name="s")

  @functools.partial(
      pl.kernel,
      mesh=mesh,
      compiler_params=pltpu.CompilerParams(use_tc_tiling_on_sc=False),
      out_type=[
          jax.ShapeDtypeStruct((B, H), jnp.float32),
          jax.ShapeDtypeStruct((B, LANES), jnp.float32),
      ],
      scratch_types=[
          pltpu.VMEM((2, CH), jnp.int32),
          pltpu.VMEM((2, CH), jnp.int32),
          pltpu.VMEM((2, CH), jnp.int32),
          pltpu.VMEM((2, CH), jnp.int32),
          pltpu.VMEM((HPAD, H), jnp.float32),
          pltpu.VMEM((HPAD, H), jnp.float32),
          pltpu.VMEM((IPW, H), jnp.float32),
          pltpu.VMEM((IPW, LANES), jnp.float32),
          pltpu.VMEM((H,), jnp.float32),
          pltpu.VMEM((LANES,), jnp.float32),
          pltpu.SemaphoreType.DMA,
          pltpu.SemaphoreType.DMA,
          pltpu.SemaphoreType.DMA,
          pltpu.SemaphoreType.DMA,
          pltpu.SemaphoreType.DMA,
          pltpu.SemaphoreType.DMA,
      ],
  )
  def k(x_hbm, ids_hbm, tab_hbm, w_hbm, b_hbm, comb_hbm, conf_hbm,
        idx0, idx1, idx2, idx3, rows0, rows1, xs, confs, wv, bv,
        isem0, isem1, isem2, isem3, gsem0, gsem1):
    wid = lax.axis_index("s") * NC + lax.axis_index("c")
    base = wid * IPW
    pltpu.sync_copy(x_hbm.at[pl.ds(base, IPW)], xs)
    pltpu.sync_copy(w_hbm, wv)
    pltpu.sync_copy(b_hbm, bv)

    idxb = (idx0, idx1, idx2, idx3)
    isem = (isem0, isem1, isem2, isem3)
    rowsb = (rows0, rows1)
    gsem = (gsem0, gsem1)

    def idx_start(li, q):
      pltpu.async_copy(ids_hbm.at[base + li], idxb[q], isem[q])

    def idx_wait(li, q):
      pltpu.make_async_copy(ids_hbm.at[base + li], idxb[q], isem[q]).wait()

    def gather_start(q, p):
      pltpu.async_copy(tab_hbm.at[idxb[q].at[0]],
                       rowsb[p].at[pl.ds(0, CH)], gsem[p])
      pltpu.async_copy(tab_hbm.at[idxb[q].at[1]],
                       rowsb[p].at[pl.ds(CH, CH)], gsem[p])

    def gather_wait(q, p):
      pltpu.make_async_copy(tab_hbm.at[idxb[q].at[0]],
                            rowsb[p].at[pl.ds(0, CH)], gsem[p]).wait()
      pltpu.make_async_copy(tab_hbm.at[idxb[q].at[1]],
                            rowsb[p].at[pl.ds(CH, CH)], gsem[p]).wait()

    for u in range(4):
      idx_start(u, u)
    for u in range(2):
      idx_wait(u, u)
      gather_start(u, u)

    wvec = [wv[pl.ds(d * LANES, LANES)] for d in range(NVD)]
    bvec = bv[...]
    inv = 1.0 / HIST
    lanes = lax.iota(jnp.int32, LANES)

    dnums = lax.GatherDimensionNumbers(
        offset_dims=(), collapsed_slice_dims=(0,), start_index_map=(0,))

    def lane_sum(v):
      for s in (8, 4, 2, 1):
        perm = lax.gather(v, (lanes ^ s)[:, None], dnums, slice_sizes=(1,),
                          unique_indices=True, indices_are_sorted=False,
                          mode=lax.GatherScatterMode.PROMISE_IN_BOUNDS)
        v = v + perm
      return v

    def item(li, q, p):
      gather_wait(q, p)

      @pl.when(li + 4 < IPW)
      def _():
        idx_start(li + 4, q)

      rb = rowsb[p]

      def step(j, accs):
        new = list(accs)
        for uu in range(4):
          r = j * 4 + uu
          for d in range(NVD):
            new[d] = new[d] + rb[r, pl.ds(d * LANES, LANES)]
        return tuple(new)

      init = tuple(jnp.zeros((LANES,), jnp.float32) for _ in range(NVD))
      accs = lax.fori_loop(0, HIST // 4, step, init)

      z = None
      for d in range(NVD):
        c = xs[li, pl.ds(d * LANES, LANES)] + accs[d] * inv
        xs[li, pl.ds(d * LANES, LANES)] = c
        t = c * wvec[d]
        z = t if z is None else z + t
      zv = lane_sum(z) + bvec
      confs[li] = 1.0 / (1.0 + jnp.exp(-zv))

      @pl.when(li + 2 < IPW)
      def _():
        idx_wait(li + 2, (q + 2) % 4)
        gather_start((q + 2) % 4, p)

    def outer(g, carry):
      for u in range(4):
        item(4 * g + u, u, u % 2)
      return carry

    lax.fori_loop(0, IPW // 4, outer, 0)

    pltpu.sync_copy(xs, comb_hbm.at[pl.ds(base, IPW)])
    pltpu.sync_copy(confs, conf_hbm.at[pl.ds(base, IPW)])

  return k


def kernel(x, rule_ids, rule_table, W, b):
  B = x.shape[0]
  try:
    info = plsc.get_sparse_core_info()
    nc, ns = info.num_cores, info.num_subcores
  except Exception:
    nc, ns = 2, 16
  ids = rule_ids.astype(jnp.int32)
  ids = jnp.pad(ids, ((0, 0), (0, HPAD - HIST)))
  ids3 = ids.reshape(B, 2, CH)
  w_flat = W.reshape(H).astype(jnp.float32)
  b_vec = jnp.broadcast_to(b.astype(jnp.float32), (LANES,))
  comb, conf = _build(B, nc, ns)(x, ids3, rule_table, w_flat, b_vec)
  return (comb, conf[:, :1])


# speedup vs baseline: 1.9387x; 1.9387x over previous
import functools

import jax
import jax.numpy as jnp
from jax import lax
from jax.experimental import pallas as pl
from jax.experimental.pallas import tpu as pltpu
from jax.experimental.pallas import tpu_sc as plsc

H = 64
HIST = 200
CH0 = 104
CH1 = HIST - CH0
LANES = 16
NVD = H // LANES
NB = 4
NBI = 2 * NB


def _build(B, NC, NS):
  NW = NC * NS
  IPW = B // NW

  mesh = plsc.VectorSubcoreMesh(core_axis_name="c", subcore_axis_name="s")

  @functools.partial(
      pl.kernel,
      mesh=mesh,
      compiler_params=pltpu.CompilerParams(use_tc_tiling_on_sc=False),
      out_type=[
          jax.ShapeDtypeStruct((B, H), jnp.float32),
          jax.ShapeDtypeStruct((B, LANES), jnp.float32),
      ],
      scratch_types=[
          pltpu.VMEM((NBI, HIST), jnp.int32),
          pltpu.VMEM((NB, HIST, H), jnp.float32),
          pltpu.VMEM((IPW, H), jnp.float32),
          pltpu.VMEM((IPW, LANES), jnp.float32),
          pltpu.VMEM((H,), jnp.float32),
          pltpu.VMEM((LANES,), jnp.float32),
          pltpu.SemaphoreType.DMA((NBI,)),
          pltpu.SemaphoreType.DMA((NB,)),
      ],
  )
  def k(x_hbm, ids_hbm, tab_hbm, w_hbm, b_hbm, comb_hbm, conf_hbm,
        idxb, rowsb, xs, confs, wv, bv, isem, gsem):
    wid = lax.axis_index("s") * NC + lax.axis_index("c")
    base = wid * IPW
    pltpu.sync_copy(x_hbm.at[pl.ds(base, IPW)], xs)
    pltpu.sync_copy(w_hbm, wv)
    pltpu.sync_copy(b_hbm, bv)

    def idx_start(li, q):
      pltpu.async_copy(ids_hbm.at[base + li], idxb.at[q], isem.at[q])

    def idx_wait(li, q):
      pltpu.make_async_copy(ids_hbm.at[base + li], idxb.at[q],
                            isem.at[q]).wait()

    def gather_start(q, p):
      pltpu.async_copy(tab_hbm.at[idxb.at[q, pl.ds(0, CH0)]],
                       rowsb.at[p, pl.ds(0, CH0)], gsem.at[p])
      pltpu.async_copy(tab_hbm.at[idxb.at[q, pl.ds(CH0, CH1)]],
                       rowsb.at[p, pl.ds(CH0, CH1)], gsem.at[p])

    def gather_wait(q, p):
      pltpu.make_async_copy(tab_hbm.at[idxb.at[q, pl.ds(0, CH0)]],
                            rowsb.at[p, pl.ds(0, CH0)], gsem.at[p]).wait()
      pltpu.make_async_copy(tab_hbm.at[idxb.at[q, pl.ds(CH0, CH1)]],
                            rowsb.at[p, pl.ds(CH0, CH1)], gsem.at[p]).wait()

    for li in range(NBI):
      idx_start(li, li)
    for li in range(NB):
      idx_wait(li, li)
      gather_start(li, li)

    wvec = [wv[pl.ds(d * LANES, LANES)] for d in range(NVD)]
    bvec = bv[...]
    inv = 1.0 / HIST
    lanes = lax.iota(jnp.int32, LANES)
    dnums = lax.GatherDimensionNumbers(
        offset_dims=(), collapsed_slice_dims=(0,), start_index_map=(0,))

    def lane_sum(v):
      for s in (8, 4, 2, 1):
        perm = lax.gather(v, (lanes ^ s)[:, None], dnums, slice_sizes=(1,),
                          unique_indices=True, indices_are_sorted=False,
                          mode=lax.GatherScatterMode.PROMISE_IN_BOUNDS)
        v = v + perm
      return v

    def item(li, q, p):
      gather_wait(q, p)

      @pl.when(li + NBI < IPW)
      def _():
        idx_start(li + NBI, q)

      def step(j, accs):
        new = list(accs)
        for uu in range(4):
          r = j * 4 + uu
          for d in range(NVD):
            new[d] = new[d] + rowsb[p, r, pl.ds(d * LANES, LANES)]
        return tuple(new)

      init = tuple(jnp.zeros((LANES,), jnp.float32) for _ in range(NVD))
      accs = lax.fori_loop(0, HIST // 4, step, init)

      z = None
      for d in range(NVD):
        c = xs[li, pl.ds(d * LANES, LANES)] + accs[d] * inv
        xs[li, pl.ds(d * LANES, LANES)] = c
        t = c * wvec[d]
        z = t if z is None else z + t
      zv = lane_sum(z) + bvec
      confs[li] = 1.0 / (1.0 + jnp.exp(-zv))

      @pl.when(li + NB < IPW)
      def _():
        idx_wait(li + NB, (q + NB) % NBI)
        gather_start((q + NB) % NBI, p)

    def outer(g, carry):
      for u in range(NBI):
        item(NBI * g + u, u, u % NB)
      return carry

    lax.fori_loop(0, IPW // NBI, outer, 0)

    pltpu.sync_copy(xs, comb_hbm.at[pl.ds(base, IPW)])
    pltpu.sync_copy(confs, conf_hbm.at[pl.ds(base, IPW)])

  return k


def kernel(x, rule_ids, rule_table, W, b):
  B = x.shape[0]
  try:
    info = plsc.get_sparse_core_info()
    nc, ns = info.num_cores, info.num_subcores
  except Exception:
    nc, ns = 2, 16
  ids = rule_ids.astype(jnp.int32)
  w_flat = W.reshape(H).astype(jnp.float32)
  b_vec = jnp.broadcast_to(b.astype(jnp.float32), (LANES,))
  comb, conf = _build(B, nc, ns)(x, ids, rule_table, w_flat, b_vec)
  return (comb, conf[:, :1])
